# baseline (device time: 22007 ns/iter reference)
import jax
import jax.numpy as jnp
from jax import lax
from jax.experimental import pallas as pl
from jax.experimental.pallas import tpu as pltpu

N_DEV = 8
BM = 512


def kernel(x, dy, gamma):
    m, d = x.shape
    n_blk = m // BM

    def body(x_ref, dy_ref, gamma_ref, out_ref, acc_ref, comm_ref,
             send_sems, recv_sems):
        my_pos = lax.axis_index("i")
        i = pl.program_id(0)
        barrier_sem = pltpu.get_barrier_semaphore()

        @pl.when(i == 0)
        def _():
            for k in range(1, N_DEV):
                peer = lax.rem(my_pos + k, N_DEV)
                pl.semaphore_signal(
                    barrier_sem, inc=1,
                    device_id=(peer,), device_id_type=pl.DeviceIdType.MESH,
                )

        xv = x_ref[:, :]
        dyv = dy_ref[:, :]
        inv_d = 1.0 / d
        sx = jnp.sum(xv, axis=1, keepdims=True)
        sxx = jnp.sum(xv * xv, axis=1, keepdims=True)
        mu = sx * inv_d
        var = sxx * inv_d - mu * mu
        a = lax.rsqrt(var + 1e-5)
        b = mu * a
        dgamma = jnp.sum(dyv * (xv * a - b), axis=0)
        dbeta = jnp.sum(dyv, axis=0)
        partial = jnp.stack([dgamma, dbeta])

        @pl.when(i == 0)
        def _():
            acc_ref[:, :] = partial

        @pl.when(i > 0)
        def _():
            acc_ref[:, :] = acc_ref[:, :] + partial

        @pl.when(i == n_blk - 1)
        def _():
            comm_ref[pl.ds(my_pos, 1)] = acc_ref[:, :][None]
            pl.semaphore_wait(barrier_sem, N_DEV - 1)

            sends = []
            for k in range(1, N_DEV):
                peer = lax.rem(my_pos + k, N_DEV)
                rdma = pltpu.make_async_remote_copy(
                    src_ref=comm_ref.at[pl.ds(my_pos, 1)],
                    dst_ref=comm_ref.at[pl.ds(my_pos, 1)],
                    send_sem=send_sems.at[k - 1],
                    recv_sem=recv_sems.at[my_pos],
                    device_id=(peer,),
                    device_id_type=pl.DeviceIdType.MESH,
                )
                rdma.start()
                sends.append(rdma)

            for k in range(1, N_DEV):
                src = lax.rem(my_pos + k, N_DEV)
                recv = pltpu.make_async_remote_copy(
                    src_ref=comm_ref.at[pl.ds(src, 1)],
                    dst_ref=comm_ref.at[pl.ds(src, 1)],
                    send_sem=send_sems.at[k - 1],
                    recv_sem=recv_sems.at[src],
                    device_id=(src,),
                    device_id_type=pl.DeviceIdType.MESH,
                )
                recv.wait_recv()

            for rdma in sends:
                rdma.wait_send()

            acc = comm_ref[0]
            for s in range(1, N_DEV):
                acc = acc + comm_ref[s]
            out_ref[:, :] = acc

    return pl.pallas_call(
        body,
        grid=(n_blk,),
        out_shape=jax.ShapeDtypeStruct((2, d), jnp.float32),
        in_specs=[
            pl.BlockSpec((BM, d), lambda i: (i, 0)),
            pl.BlockSpec((BM, d), lambda i: (i, 0)),
            pl.BlockSpec((d,), lambda i: (0,)),
        ],
        out_specs=pl.BlockSpec((2, d), lambda i: (0, 0)),
        scratch_shapes=[
            pltpu.VMEM((2, d), jnp.float32),
            pltpu.VMEM((N_DEV, 2, d), jnp.float32),
            pltpu.SemaphoreType.DMA((N_DEV - 1,)),
            pltpu.SemaphoreType.DMA((N_DEV,)),
        ],
        compiler_params=pltpu.CompilerParams(
            dimension_semantics=("arbitrary",),
            collective_id=0,
        ),
    )(x, dy, gamma)
